# baseline (speedup 1.0000x reference)
import jax
import jax.numpy as jnp
from jax import lax
from jax.experimental import pallas as pl
from jax.experimental.pallas import tpu as pltpu
from jax.experimental.pallas import tpu_sc as plsc

B = 16384
H = 200
NROWS = 1000000
NPAD = 1000064
D = 64
NC = 2
NS = 16
NW = NC * NS
BPW = B // NW
LANES = 16
JT = NPAD // 128
JPW = (JT + NW - 1) // NW
CW = 512
QMAX = NPAD - CW

_MESH_KW = dict(
    mesh=plsc.VectorSubcoreMesh(core_axis_name="c", subcore_axis_name="s"),
    compiler_params=pltpu.CompilerParams(
        use_tc_tiling_on_sc=True, disable_bounds_checks=True,
        needs_layout_passes=False),
)


def _wid():
    return lax.axis_index("s") * NC + lax.axis_index("c")


def _lastid_body(hist_hbm, seqlen_hbm, lastid_hbm, seq_v, hist_v, lid_v):
    base = _wid() * BPW
    lane = lax.iota(jnp.int32, LANES)
    pltpu.sync_copy(seqlen_hbm.at[pl.ds(base, BPW)], seq_v)
    for blk in range(BPW // 128):
        pltpu.sync_copy(hist_hbm.at[:, pl.ds(base + blk * 128, 128)], hist_v)
        for grp in range(128 // LANES):
            off = blk * 128 + grp * LANES
            rows = seq_v[pl.ds(off, LANES)] - 1
            cols = grp * LANES + lane
            lid_v[pl.ds(off, LANES)] = plsc.load_gather(hist_v, [rows, cols])
    pltpu.sync_copy(lid_v, lastid_hbm.at[pl.ds(base, BPW)])


def _scan_body(tab_hbm, ids_hbm, scr_hbm,
               sbuf, bk_v, hb_v, ch0, ch1,
               sb0, sb1, sb2, sb3, sb4, sb5, sb6, sb7,
               cs0, cs1, ss0, ss1, ss2, ss3, ss4, ss5, ss6, ss7):
    w = _wid()
    lane = lax.iota(jnp.int32, LANES)
    jlo = jnp.minimum(w * JPW, JT)
    jhi = jnp.minimum(jlo + JPW, JT)
    r0 = jlo * 128
    rcnt = (jhi - jlo) * 128
    nchunks = lax.div(rcnt + CW - 1, jnp.int32(CW))

    SB = 2048
    nbk = jnp.int32(0)
    for c in range(B // SB):
        pltpu.sync_copy(ids_hbm.at[pl.ds(c * SB, SB)], sbuf)

        def bkstep(i, off, c=c):
            voff = pl.multiple_of(i * LANES, LANES)
            idv = sbuf[pl.ds(voff, LANES)]
            rrel = idv - r0
            m = (rrel >= 0) & (rrel < rcnt)
            bglob = c * SB + i * LANES + lane
            pk = jnp.left_shift(bglob, 15) | (rrel & 32767)
            plsc.store_compressed(bk_v.at[pl.ds(off, LANES)], pk, mask=m)
            cnt = jnp.max(plsc.all_reduce_population_count(m))
            return off + cnt

        nbk = lax.fori_loop(0, SB // LANES, bkstep, nbk)

    nslice = lax.div(nbk + LANES - 1, jnp.int32(LANES))

    chs = [ch0, ch1]
    csems = [cs0, cs1]
    sbs = [sb0, sb1, sb2, sb3, sb4, sb5, sb6, sb7]
    ssems = [ss0, ss1, ss2, ss3, ss4, ss5, ss6, ss7]
    dump = jnp.broadcast_to(jnp.int32(B), (LANES,))

    def chunk_base(c):
        return pl.multiple_of(
            jnp.minimum(r0 + c * CW, QMAX) & ~jnp.int32(127), 128)

    def enq(c, p):
        cb = chunk_base(c)
        for t in range(CW // 128):
            pltpu.async_copy(
                tab_hbm.at[:, pl.ds(cb + t * 128, 128)],
                chs[p].at[:, pl.ds(t * 128, 128)], csems[p])

    def chwait(p):
        for t in range(CW // 128):
            pltpu.make_async_copy(
                tab_hbm.at[:, pl.ds(0, 128)],
                chs[p].at[:, pl.ds(t * 128, 128)], csems[p]).wait()

    def scwait(q):
        pltpu.make_async_copy(sbs[q], scr_hbm.at[dump], ssems[q]).wait()

    for q in range(8):
        pltpu.async_copy(sbs[q], scr_hbm.at[dump], ssems[q])
    enq(jnp.int32(0), 0)

    def extract_group(g, q, nh, p):
        @pl.when(g * LANES < nh)
        def _():
            voff = pl.multiple_of(g * LANES, LANES)
            pk2 = hb_v[pl.ds(voff, LANES)]
            cvec = pk2 & 511
            bvec = lax.shift_right_logical(pk2, 9)
            gm = (g * LANES + lane) < nh
            bsc = jnp.where(gm, bvec, jnp.int32(B))
            scwait(q)
            for d in range(D):
                vals = plsc.load_gather(
                    chs[p], [jnp.full((LANES,), d, jnp.int32), cvec])
                plsc.store_scatter(
                    sbs[q], [lane, jnp.full((LANES,), d, jnp.int32)], vals)
            pltpu.async_copy(sbs[q], scr_hbm.at[bsc], ssems[q])

    def process(c, p):
        enq(c + 1, 1 - p)
        chwait(p)
        clo = chunk_base(c) - r0

        def fstep(i, hoff):
            voff = pl.multiple_of(i * LANES, LANES)
            pk = bk_v[pl.ds(voff, LANES)]
            rrel = pk & 32767
            bv = lax.shift_right_logical(pk, 15)
            cpos = rrel - clo
            m = (cpos >= 0) & (cpos < CW) & ((i * LANES + lane) < nbk)
            pk2 = jnp.left_shift(bv, 9) | (cpos & 511)
            plsc.store_compressed(hb_v.at[pl.ds(hoff, LANES)], pk2, mask=m)
            cnt = jnp.max(plsc.all_reduce_population_count(m))
            return hoff + cnt

        nh = lax.fori_loop(0, nslice, fstep, jnp.int32(0))
        npairs = lax.div(nh + 4 * LANES - 1, jnp.int32(4 * LANES))

        def pairstep(t, carry):
            for u in range(4):
                extract_group(4 * t + u, 4 * p + u, nh, p)
            return carry

        lax.fori_loop(0, npairs, pairstep, jnp.int32(0))

    def do_chunk(c, carry):
        @pl.when(lax.rem(c, 2) == 0)
        def _():
            process(c, 0)

        @pl.when(lax.rem(c, 2) == 1)
        def _():
            process(c, 1)

        return carry

    lax.fori_loop(0, nchunks, do_chunk, jnp.int32(0))

    @pl.when(lax.rem(nchunks, 2) == 0)
    def _():
        chwait(0)

    @pl.when(lax.rem(nchunks, 2) == 1)
    def _():
        chwait(1)

    for q in range(8):
        scwait(q)


def _comb_body(su_hbm, si_hbm, gemb_hbm, out_hbm, su_v, si_v, ov_v, gemb_v):
    base = _wid() * BPW
    pltpu.sync_copy(gemb_hbm, gemb_v)
    gvec = [gemb_v[pl.ds(k * LANES, LANES)] for k in range(D // LANES)]
    for t in range(BPW // 128):
        rb = base + t * 128
        pltpu.sync_copy(su_hbm.at[pl.ds(rb, 128), :], su_v)
        pltpu.sync_copy(si_hbm.at[pl.ds(rb, 128), :], si_v)

        def rstep(r, carry):
            for k in range(D // LANES):
                s = pl.ds(k * LANES, LANES)
                ov_v[r, s] = su_v[r, s] + si_v[r, s] + gvec[k]
            return carry

        lax.fori_loop(0, 128, rstep, jnp.int32(0))
        pltpu.sync_copy(ov_v, out_hbm.at[pl.ds(rb, 128)])


@jax.jit
def kernel(in_item_id, seqlen, user_id, user_table, item_table,
           global_user_emb):
    ut = user_table.T
    it = item_table.T
    hist = in_item_id.T

    lastid = pl.kernel(
        _lastid_body,
        out_type=jax.ShapeDtypeStruct((B,), jnp.int32),
        scratch_types=[
            pltpu.VMEM((BPW,), jnp.int32),
            pltpu.VMEM((H, 128), jnp.int32),
            pltpu.VMEM((BPW,), jnp.int32),
        ],
        **_MESH_KW,
    )(hist, seqlen)

    scan_scratch = [
        pltpu.VMEM((2048,), jnp.int32),
        pltpu.VMEM((B + LANES,), jnp.int32),
        pltpu.VMEM((B + LANES,), jnp.int32),
        pltpu.VMEM((D, CW), jnp.float32),
        pltpu.VMEM((D, CW), jnp.float32),
    ] + [pltpu.VMEM((LANES, 128), jnp.float32)] * 8 \
      + [pltpu.SemaphoreType.DMA] * 10
    scr_u = pl.kernel(
        _scan_body,
        out_type=jax.ShapeDtypeStruct((B + LANES, 128), jnp.float32),
        scratch_types=scan_scratch,
        **_MESH_KW,
    )(ut, user_id)
    scr_i = pl.kernel(
        _scan_body,
        out_type=jax.ShapeDtypeStruct((B + LANES, 128), jnp.float32),
        scratch_types=scan_scratch,
        **_MESH_KW,
    )(it, lastid)

    return pl.kernel(
        _comb_body,
        out_type=jax.ShapeDtypeStruct((B, D), jnp.float32),
        scratch_types=[
            pltpu.VMEM((128, 128), jnp.float32),
            pltpu.VMEM((128, 128), jnp.float32),
            pltpu.VMEM((128, D), jnp.float32),
            pltpu.VMEM((D,), jnp.float32),
        ],
        **_MESH_KW,
    )(scr_u, scr_i, global_user_emb)


# speedup vs baseline: 5.3289x; 5.3289x over previous
import jax
import jax.numpy as jnp
from jax import lax
from jax.experimental import pallas as pl
from jax.experimental.pallas import tpu as pltpu
from jax.experimental.pallas import tpu_sc as plsc

B = 16384
H = 200
NUSERS = 1000000
D = 64
NC = 2
NS = 16
NW = NC * NS
BPW = B // NW
LANES = 16
HCH = 128
NSLOT = 4


def _extract_scalar(vec, lane, l):
    return jnp.max(jnp.where(lane == l, vec, 0))


def _body(hist_hbm, seqlen_hbm, uid_hbm, ut_hbm, it_hbm, gemb_hbm,
          out_hbm,
          seq_v, uid_v, lastid_v, hist_v, gemb_v, out_v,
          *bufsems):
    ubufs = bufsems[:NSLOT]
    ibufs = bufsems[NSLOT:2 * NSLOT]
    usems = bufsems[2 * NSLOT:3 * NSLOT]
    isems = bufsems[3 * NSLOT:4 * NSLOT]

    wid = lax.axis_index("s") * NC + lax.axis_index("c")
    base = wid * BPW
    lane = lax.iota(jnp.int32, LANES)

    pltpu.sync_copy(seqlen_hbm.at[pl.ds(base, BPW)], seq_v)
    pltpu.sync_copy(uid_hbm.at[pl.ds(base, BPW)], uid_v)
    pltpu.sync_copy(gemb_hbm, gemb_v)
    gvec = [gemb_v[pl.ds(k * LANES, LANES)] for k in range(D // LANES)]

    for blk in range(BPW // HCH):
        pltpu.sync_copy(hist_hbm.at[:, pl.ds(base + blk * HCH, HCH)], hist_v)
        for grp in range(HCH // LANES):
            off = blk * HCH + grp * LANES
            rows = seq_v[pl.ds(off, LANES)] - 1
            cols = grp * LANES + lane
            lastid_v[pl.ds(off, LANES)] = plsc.load_gather(
                hist_v, [rows, cols])

    def _fetch(r, s):
        g16 = lax.shift_right_logical(r, 4)
        voff = pl.multiple_of(g16 * LANES, LANES)
        l = r - g16 * LANES
        ru = _extract_scalar(uid_v[pl.ds(voff, LANES)], lane, l)
        ri = _extract_scalar(lastid_v[pl.ds(voff, LANES)], lane, l)
        ju = pl.multiple_of(lax.shift_right_logical(ru, 7) * 128, 128)
        ji = pl.multiple_of(lax.shift_right_logical(ri, 7) * 128, 128)
        pltpu.async_copy(ut_hbm.at[:, pl.ds(ju, 128)], ubufs[s], usems[s])
        pltpu.async_copy(it_hbm.at[:, pl.ds(ji, 128)], ibufs[s], isems[s])
        return ru, ri

    def _consume(r, hbase, s, ru, ri):
        cu = jnp.broadcast_to(ru & 127, (LANES,))
        ci = jnp.broadcast_to(ri & 127, (LANES,))
        pltpu.make_async_copy(
            ut_hbm.at[:, pl.ds(0, 128)], ubufs[s], usems[s]).wait()
        pltpu.make_async_copy(
            it_hbm.at[:, pl.ds(0, 128)], ibufs[s], isems[s]).wait()
        for k in range(D // LANES):
            dvec = k * LANES + lane
            uval = plsc.load_gather(ubufs[s], [dvec, cu])
            ival = plsc.load_gather(ibufs[s], [dvec, ci])
            out_v[r - hbase, pl.ds(k * LANES, LANES)] = uval + ival + gvec[k]

    HB = BPW // 2
    for half in range(2):
        hbase = half * HB
        carry0 = []
        for s in range(NSLOT):
            carry0.extend(_fetch(jnp.int32(hbase + s), s))

        def step(it, carry, hbase=hbase):
            rbase = hbase + NSLOT * it
            out = []
            for s in range(NSLOT):
                _consume(rbase + s, hbase, s, carry[2 * s], carry[2 * s + 1])
                nxt = hbase + lax.rem(rbase + s + NSLOT - hbase, HB)
                out.extend(_fetch(nxt, s))
            return tuple(out)

        lax.fori_loop(0, HB // NSLOT, step, tuple(carry0))

        for s in range(NSLOT):
            pltpu.make_async_copy(
                ut_hbm.at[:, pl.ds(0, 128)], ubufs[s], usems[s]).wait()
            pltpu.make_async_copy(
                it_hbm.at[:, pl.ds(0, 128)], ibufs[s], isems[s]).wait()

        pltpu.sync_copy(out_v, out_hbm.at[pl.ds(base + hbase, HB)])


@jax.jit
def kernel(in_item_id, seqlen, user_id, user_table, item_table,
           global_user_emb):
    ut = user_table.T
    it = item_table.T
    hist = in_item_id.T
    run = pl.kernel(
        _body,
        out_type=jax.ShapeDtypeStruct((B, D), jnp.float32),
        mesh=plsc.VectorSubcoreMesh(core_axis_name="c", subcore_axis_name="s"),
        compiler_params=pltpu.CompilerParams(
            use_tc_tiling_on_sc=True, disable_bounds_checks=True,
            needs_layout_passes=False),
        scratch_types=[
            pltpu.VMEM((BPW,), jnp.int32),
            pltpu.VMEM((BPW,), jnp.int32),
            pltpu.VMEM((BPW,), jnp.int32),
            pltpu.VMEM((H, HCH), jnp.int32),
            pltpu.VMEM((D,), jnp.float32),
            pltpu.VMEM((BPW // 2, D), jnp.float32),
        ] + [pltpu.VMEM((D, 128), jnp.float32)] * (2 * NSLOT)
          + [pltpu.SemaphoreType.DMA] * (2 * NSLOT),
    )
    return run(hist, seqlen, user_id, ut, it, global_user_emb)
